# baseline (speedup 1.0000x reference)
import functools

import jax
import jax.numpy as jnp
from jax import lax
from jax.experimental import pallas as pl
from jax.experimental.pallas import tpu as pltpu
from jax.experimental.pallas import tpu_sc as plsc

_NX, _NY, _NZ = 512, 512, 1
_C = 128
_P = 60000
_CELLS = _NZ * _NY * _NX
_NW = 32
_STRIPE = _CELLS // _NW
_SUB = 256
_NSUB = _STRIPE // _SUB
_CH1 = 2000
_NCH1 = _P // _CH1
_UNROLL1 = 5
_LISTCAP = _STRIPE + 16
_OPAD = 257
_ROWCAP = _SUB + 16


def _iota16():
    return lax.iota(jnp.int32, 16)


def _body(idx_hbm, pf_hbm, out_hbm,
          w_map, ibuf, plist, dlist, rows, otile, segs,
          sem_row, sem_out0, sem_out1):
    wid = lax.axis_index("s") * 2 + lax.axis_index("c")
    base = wid * _STRIPE
    iota = _iota16()
    zerosf = jnp.zeros((16,), jnp.float32)

    def initw(i, _):
        w_map[pl.ds(i * 16, 16)] = jnp.full((16,), -1, jnp.int32)
        return 0
    lax.fori_loop(0, _STRIPE // 16, initw, 0)

    for b in range(2):
        def inito(c, _):
            for cg in range(_SUB // 16):
                otile[b, c, pl.ds(cg * 16, 16)] = zerosf
            otile[b, c, pl.ds(_OPAD - 16, 16)] = zerosf
            return 0
        lax.fori_loop(0, _C, inito, 0)

    def p1_chunk(ci, _):
        off = ci * _CH1
        pltpu.sync_copy(idx_hbm.at[pl.ds(off, _CH1)], ibuf)

        def grp(g, _):
            for u in range(_UNROLL1):
                o = g * (16 * _UNROLL1) + u * 16
                idx = ibuf[pl.ds(o, 16)]
                lidx = idx - base
                inr = (lidx >= 0) & (lidx < _STRIPE)
                _cnt, lastm = plsc.scan_count(idx)
                m = lastm & inr
                plsc.store_scatter(w_map, [lidx], off + o + iota, mask=m)
            return 0
        lax.fori_loop(0, _CH1 // (16 * _UNROLL1), grp, 0)
        return 0
    lax.fori_loop(0, _NCH1, p1_chunk, 0)

    def comp(s, cur):
        segs[s] = cur

        def cgrp(j, cur2):
            w = w_map[pl.ds(s * _SUB + j * 16, 16)]
            m = w >= 0
            plsc.store_compressed(plist.at[pl.ds(cur2, 16)], w, mask=m)
            plsc.store_compressed(dlist.at[pl.ds(cur2, 16)],
                                  j * 16 + iota, mask=m)
            return cur2 + plsc.all_reduce_population_count(m)[0]
        return lax.fori_loop(0, _SUB // 16, cgrp, cur)
    ktot = lax.fori_loop(0, _NSUB, comp, jnp.int32(0))
    segs[_NSUB] = ktot
    plsc.store_compressed(plist.at[pl.ds(ktot, 16)],
                          jnp.zeros((16,), jnp.int32),
                          mask=jnp.full((16,), True))

    def half(s, b, pseg, sem_out):
        pseg0, pseg1 = pseg
        col0 = base + s * _SUB

        @pl.when(s >= 2)
        def _wait_prev():
            pltpu.make_async_copy(
                otile.at[b, :, 0:_SUB],
                out_hbm.at[:, pl.ds(col0, _SUB)], sem_out).wait()

        def rgrp(g, _):
            dvec = dlist[pl.ds(g * 16, 16)]
            for i in range(16):
                k = g * 16 + i
                m = jnp.broadcast_to((k >= pseg0) & (k < pseg1), (16,))
                dloc = dvec[i]
                for cg in range(_C // 16):
                    plsc.store_scatter(
                        otile,
                        [jnp.full((16,), b, jnp.int32), cg * 16 + iota,
                         jnp.broadcast_to(dloc, (16,))],
                        zerosf, mask=m)
            return 0
        lax.fori_loop(pseg0 >> 4, (pseg1 + 15) >> 4, rgrp, 0)

        seg0 = segs[s]
        seg1 = segs[s + 1]
        ga = seg0 >> 4
        gb = (seg1 + 15) >> 4

        def fire(g, _):
            pvec = plist[pl.ds(g * 16, 16)]
            pltpu.make_async_copy(
                pf_hbm.at[pvec],
                rows.at[pl.ds((g - ga) * 16, 16), :], sem_row).start()
            return 0
        lax.fori_loop(ga, gb, fire, 0)

        def drain(g, _):
            pvec = plist[pl.ds(0, 16)]
            pltpu.make_async_copy(
                pf_hbm.at[pvec], rows.at[pl.ds(0, 16), :], sem_row).wait()
            return 0
        lax.fori_loop(ga, gb, drain, 0)

        def agrp(g, _):
            dvec = dlist[pl.ds(g * 16, 16)]
            for i in range(16):
                k = g * 16 + i
                m = jnp.broadcast_to((k >= seg0) & (k < seg1), (16,))
                dloc = dvec[i]
                krow = (g - ga) * 16 + i
                for cg in range(_C // 16):
                    vec = rows[krow, pl.ds(cg * 16, 16)]
                    plsc.store_scatter(
                        otile,
                        [jnp.full((16,), b, jnp.int32), cg * 16 + iota,
                         jnp.broadcast_to(dloc, (16,))],
                        vec, mask=m)
            return 0
        lax.fori_loop(ga, gb, agrp, 0)

        pltpu.make_async_copy(
            otile.at[b, :, 0:_SUB],
            out_hbm.at[:, pl.ds(col0, _SUB)], sem_out).start()
        return (seg0, seg1)

    def pair(sp, carry):
        pa, pb = carry
        pa = half(sp * 2, 0, pa, sem_out0)
        pb = half(sp * 2 + 1, 1, pb, sem_out1)
        return (pa, pb)

    z = jnp.int32(0)
    lax.fori_loop(0, _NSUB // 2, pair, ((z, z), (z, z)))

    pltpu.make_async_copy(
        otile.at[0, :, 0:_SUB],
        out_hbm.at[:, pl.ds(base, _SUB)], sem_out0).wait()
    pltpu.make_async_copy(
        otile.at[1, :, 0:_SUB],
        out_hbm.at[:, pl.ds(base, _SUB)], sem_out1).wait()


_mesh = plsc.VectorSubcoreMesh(core_axis_name="c", subcore_axis_name="s")

_scatter = functools.partial(
    pl.kernel,
    out_type=jax.ShapeDtypeStruct((_C, _CELLS), jnp.float32),
    mesh=_mesh,
    compiler_params=pltpu.CompilerParams(use_tc_tiling_on_sc=False,
                                         needs_layout_passes=False,
                                         disable_bounds_checks=True,
                                         disable_semaphore_checks=True,
                                         skip_device_barrier=True),
    scratch_types=[
        pltpu.VMEM((_STRIPE,), jnp.int32),
        pltpu.VMEM((_CH1,), jnp.int32),
        pltpu.VMEM((_LISTCAP,), jnp.int32),
        pltpu.VMEM((_LISTCAP,), jnp.int32),
        pltpu.VMEM((_ROWCAP, _C), jnp.float32),
        pltpu.VMEM((2, _C, _OPAD), jnp.float32),
        pltpu.SMEM((_NSUB + 2,), jnp.int32),
        pltpu.SemaphoreType.DMA,
        pltpu.SemaphoreType.DMA,
        pltpu.SemaphoreType.DMA,
    ],
)(_body)


def kernel(pillar_features, coords):
    ci = coords.astype(jnp.int32)
    idx = ci[:, 1] * (_NY * _NX) + ci[:, 2] * _NX + ci[:, 3]
    pf = pillar_features.astype(jnp.float32)
    out = _scatter(idx, pf)
    return out.reshape(1, _C * _NZ, _NY, _NX)


# speedup vs baseline: 1.0004x; 1.0004x over previous
import functools

import jax
import jax.numpy as jnp
from jax import lax
from jax.experimental import pallas as pl
from jax.experimental.pallas import tpu as pltpu
from jax.experimental.pallas import tpu_sc as plsc

_NX, _NY, _NZ = 512, 512, 1
_C = 128
_P = 60000
_CELLS = _NZ * _NY * _NX
_NW = 32
_STRIPE = _CELLS // _NW
_SUB = 256
_NSUB = _STRIPE // _SUB
_CH1 = 2000
_NCH1 = _P // _CH1
_UNROLL1 = 5
_LISTCAP = _STRIPE + 16
_OPAD = 257
_ROWCAP = _SUB + 16


def _iota16():
    return lax.iota(jnp.int32, 16)


def _body(idx_hbm, pf_hbm, out_hbm,
          w_map, ibuf, plist, dlist, rows, otile, segs,
          sem_row, sem_out0, sem_out1):
    wid = lax.axis_index("s") * 2 + lax.axis_index("c")
    base = wid * _STRIPE
    iota = _iota16()
    zerosf = jnp.zeros((16,), jnp.float32)

    _scope_init = jax.named_scope("ph0_init"); _scope_init.__enter__()
    def initw(i, _):
        w_map[pl.ds(i * 16, 16)] = jnp.full((16,), -1, jnp.int32)
        return 0
    lax.fori_loop(0, _STRIPE // 16, initw, 0)

    for b in range(2):
        def inito(c, _):
            for cg in range(_SUB // 16):
                otile[b, c, pl.ds(cg * 16, 16)] = zerosf
            otile[b, c, pl.ds(_OPAD - 16, 16)] = zerosf
            return 0
        lax.fori_loop(0, _C, inito, 0)

    _scope_init.__exit__(None, None, None)
    _scope_p1 = jax.named_scope("ph1_winner"); _scope_p1.__enter__()
    def p1_chunk(ci, _):
        off = ci * _CH1
        pltpu.sync_copy(idx_hbm.at[pl.ds(off, _CH1)], ibuf)

        def grp(g, _):
            for u in range(_UNROLL1):
                o = g * (16 * _UNROLL1) + u * 16
                idx = ibuf[pl.ds(o, 16)]
                lidx = idx - base
                inr = (lidx >= 0) & (lidx < _STRIPE)
                _cnt, lastm = plsc.scan_count(idx)
                m = lastm & inr
                plsc.store_scatter(w_map, [lidx], off + o + iota, mask=m)
            return 0
        lax.fori_loop(0, _CH1 // (16 * _UNROLL1), grp, 0)
        return 0
    lax.fori_loop(0, _NCH1, p1_chunk, 0)

    _scope_p1.__exit__(None, None, None)
    _scope_p2 = jax.named_scope("ph2_compact"); _scope_p2.__enter__()
    def comp(s, cur):
        segs[s] = cur

        def cgrp(j, cur2):
            w = w_map[pl.ds(s * _SUB + j * 16, 16)]
            m = w >= 0
            plsc.store_compressed(plist.at[pl.ds(cur2, 16)], w, mask=m)
            plsc.store_compressed(dlist.at[pl.ds(cur2, 16)],
                                  j * 16 + iota, mask=m)
            return cur2 + plsc.all_reduce_population_count(m)[0]
        return lax.fori_loop(0, _SUB // 16, cgrp, cur)
    ktot = lax.fori_loop(0, _NSUB, comp, jnp.int32(0))
    segs[_NSUB] = ktot
    plsc.store_compressed(plist.at[pl.ds(ktot, 16)],
                          jnp.zeros((16,), jnp.int32),
                          mask=jnp.full((16,), True))

    _scope_p2.__exit__(None, None, None)
    _scope_p3 = jax.named_scope("ph3_assemble"); _scope_p3.__enter__()
    def half(s, b, pseg, sem_out):
        pseg0, pseg1 = pseg
        col0 = base + s * _SUB

        @pl.when(s >= 2)
        def _wait_prev():
            pltpu.make_async_copy(
                otile.at[b, :, 0:_SUB],
                out_hbm.at[:, pl.ds(col0, _SUB)], sem_out).wait()

        def rgrp(g, _):
            dvec = dlist[pl.ds(g * 16, 16)]
            for i in range(16):
                k = g * 16 + i
                m = jnp.broadcast_to((k >= pseg0) & (k < pseg1), (16,))
                dloc = dvec[i]
                for cg in range(_C // 16):
                    plsc.store_scatter(
                        otile,
                        [jnp.full((16,), b, jnp.int32), cg * 16 + iota,
                         jnp.broadcast_to(dloc, (16,))],
                        zerosf, mask=m)
            return 0
        lax.fori_loop(pseg0 >> 4, (pseg1 + 15) >> 4, rgrp, 0)

        seg0 = segs[s]
        seg1 = segs[s + 1]
        ga = seg0 >> 4
        gb = (seg1 + 15) >> 4

        def fire(g, _):
            pvec = plist[pl.ds(g * 16, 16)]
            pltpu.make_async_copy(
                pf_hbm.at[pvec],
                rows.at[pl.ds((g - ga) * 16, 16), :], sem_row).start()
            return 0
        lax.fori_loop(ga, gb, fire, 0)

        def drain(g, _):
            pvec = plist[pl.ds(0, 16)]
            pltpu.make_async_copy(
                pf_hbm.at[pvec], rows.at[pl.ds(0, 16), :], sem_row).wait()
            return 0
        lax.fori_loop(ga, gb, drain, 0)

        def agrp(g, _):
            dvec = dlist[pl.ds(g * 16, 16)]
            for i in range(16):
                k = g * 16 + i
                m = jnp.broadcast_to((k >= seg0) & (k < seg1), (16,))
                dloc = dvec[i]
                krow = (g - ga) * 16 + i
                for cg in range(_C // 16):
                    vec = rows[krow, pl.ds(cg * 16, 16)]
                    plsc.store_scatter(
                        otile,
                        [jnp.full((16,), b, jnp.int32), cg * 16 + iota,
                         jnp.broadcast_to(dloc, (16,))],
                        vec, mask=m)
            return 0
        lax.fori_loop(ga, gb, agrp, 0)

        pltpu.make_async_copy(
            otile.at[b, :, 0:_SUB],
            out_hbm.at[:, pl.ds(col0, _SUB)], sem_out).start()
        return (seg0, seg1)

    def pair(sp, carry):
        pa, pb = carry
        pa = half(sp * 2, 0, pa, sem_out0)
        pb = half(sp * 2 + 1, 1, pb, sem_out1)
        return (pa, pb)

    z = jnp.int32(0)
    lax.fori_loop(0, _NSUB // 2, pair, ((z, z), (z, z)))

    _scope_p3.__exit__(None, None, None)
    pltpu.make_async_copy(
        otile.at[0, :, 0:_SUB],
        out_hbm.at[:, pl.ds(base, _SUB)], sem_out0).wait()
    pltpu.make_async_copy(
        otile.at[1, :, 0:_SUB],
        out_hbm.at[:, pl.ds(base, _SUB)], sem_out1).wait()


_mesh = plsc.VectorSubcoreMesh(core_axis_name="c", subcore_axis_name="s")

_scatter = functools.partial(
    pl.kernel,
    out_type=jax.ShapeDtypeStruct((_C, _CELLS), jnp.float32),
    mesh=_mesh,
    compiler_params=pltpu.CompilerParams(use_tc_tiling_on_sc=False,
                                         needs_layout_passes=False),
    scratch_types=[
        pltpu.VMEM((_STRIPE,), jnp.int32),
        pltpu.VMEM((_CH1,), jnp.int32),
        pltpu.VMEM((_LISTCAP,), jnp.int32),
        pltpu.VMEM((_LISTCAP,), jnp.int32),
        pltpu.VMEM((_ROWCAP, _C), jnp.float32),
        pltpu.VMEM((2, _C, _OPAD), jnp.float32),
        pltpu.SMEM((_NSUB + 2,), jnp.int32),
        pltpu.SemaphoreType.DMA,
        pltpu.SemaphoreType.DMA,
        pltpu.SemaphoreType.DMA,
    ],
)(_body)


def kernel(pillar_features, coords):
    ci = coords.astype(jnp.int32)
    idx = ci[:, 1] * (_NY * _NX) + ci[:, 2] * _NX + ci[:, 3]
    pf = pillar_features.astype(jnp.float32)
    out = _scatter(idx, pf)
    return out.reshape(1, _C * _NZ, _NY, _NX)


# speedup vs baseline: 1.1896x; 1.1891x over previous
import functools

import jax
import jax.numpy as jnp
from jax import lax
from jax.experimental import pallas as pl
from jax.experimental.pallas import tpu as pltpu
from jax.experimental.pallas import tpu_sc as plsc

_NX, _NY, _NZ = 512, 512, 1
_C = 128
_P = 60000
_CELLS = _NZ * _NY * _NX
_NW = 32
_STRIPE = _CELLS // _NW
_SUB = 256
_NSUB = _STRIPE // _SUB
_CH1 = 2000
_NCH1 = _P // _CH1
_UNROLL1 = 5
_LISTCAP = _STRIPE + 16
_OPAD = 257
_ROWCAP = _SUB + 16


def _iota16():
    return lax.iota(jnp.int32, 16)


def _body(idx_hbm, pf_hbm, out_hbm,
          w_map, ibuf, plist, dlist, rows, otile, segs,
          sem_row, sem_out0, sem_out1):
    wid = lax.axis_index("s") * 2 + lax.axis_index("c")
    base = wid * _STRIPE
    iota = _iota16()
    zerosf = jnp.zeros((16,), jnp.float32)

    _scope_init = jax.named_scope("ph0_init"); _scope_init.__enter__()
    def initw(i, _):
        w_map[pl.ds(i * 16, 16)] = jnp.full((16,), -1, jnp.int32)
        return 0
    lax.fori_loop(0, _STRIPE // 16, initw, 0)

    for b in range(2):
        def inito(c, _):
            for cg in range(_SUB // 16):
                otile[b, c, pl.ds(cg * 16, 16)] = zerosf
            otile[b, c, pl.ds(_OPAD - 16, 16)] = zerosf
            return 0
        lax.fori_loop(0, _C, inito, 0)

    _scope_init.__exit__(None, None, None)
    _scope_p1 = jax.named_scope("ph1_winner"); _scope_p1.__enter__()
    def p1_chunk(ci, _):
        off = ci * _CH1
        pltpu.sync_copy(idx_hbm.at[pl.ds(off, _CH1)], ibuf)

        def grp(g, _):
            os_ = [g * (16 * _UNROLL1) + u * 16 for u in range(_UNROLL1)]
            idxs = [ibuf[pl.ds(o, 16)] for o in os_]
            scans = [plsc.scan_count(ix)[1] for ix in idxs]
            for o, ix, lastm in zip(os_, idxs, scans):
                lidx = ix - base
                inr = (lidx >= 0) & (lidx < _STRIPE)
                m = lastm & inr
                plsc.store_scatter(w_map, [lidx], off + o + iota, mask=m)
            return 0
        lax.fori_loop(0, _CH1 // (16 * _UNROLL1), grp, 0)
        return 0
    lax.fori_loop(0, _NCH1, p1_chunk, 0)

    _scope_p1.__exit__(None, None, None)
    _scope_p2 = jax.named_scope("ph2_compact"); _scope_p2.__enter__()
    def comp(s, cur):
        segs[s] = cur

        def cgrp(j, cur2):
            w = w_map[pl.ds(s * _SUB + j * 16, 16)]
            m = w >= 0
            plsc.store_compressed(plist.at[pl.ds(cur2, 16)], w, mask=m)
            plsc.store_compressed(dlist.at[pl.ds(cur2, 16)],
                                  j * 16 + iota, mask=m)
            return cur2 + plsc.all_reduce_population_count(m)[0]
        return lax.fori_loop(0, _SUB // 16, cgrp, cur)
    ktot = lax.fori_loop(0, _NSUB, comp, jnp.int32(0))
    segs[_NSUB] = ktot
    plsc.store_compressed(plist.at[pl.ds(ktot, 16)],
                          jnp.zeros((16,), jnp.int32),
                          mask=jnp.full((16,), True))

    _scope_p2.__exit__(None, None, None)
    _scope_p3 = jax.named_scope("ph3_assemble"); _scope_p3.__enter__()
    def half(s, b, pseg, sem_out):
        pseg0, pseg1 = pseg
        col0 = base + s * _SUB

        @pl.when(s >= 2)
        def _wait_prev():
            pltpu.make_async_copy(
                otile.at[b, :, 0:_SUB],
                out_hbm.at[:, pl.ds(col0, _SUB)], sem_out).wait()

        def rgrp(g, _):
            dvec = dlist[pl.ds(g * 16, 16)]
            for i in range(16):
                k = g * 16 + i
                m = jnp.broadcast_to((k >= pseg0) & (k < pseg1), (16,))
                dloc = dvec[i]
                for cg in range(_C // 16):
                    plsc.store_scatter(
                        otile,
                        [jnp.full((16,), b, jnp.int32), cg * 16 + iota,
                         jnp.broadcast_to(dloc, (16,))],
                        zerosf, mask=m)
            return 0
        lax.fori_loop(pseg0 >> 4, (pseg1 + 15) >> 4, rgrp, 0)

        seg0 = segs[s]
        seg1 = segs[s + 1]
        ga = seg0 >> 4
        gb = (seg1 + 15) >> 4

        def fire(g, _):
            pvec = plist[pl.ds(g * 16, 16)]
            pltpu.make_async_copy(
                pf_hbm.at[pvec],
                rows.at[pl.ds((g - ga) * 16, 16), :], sem_row).start()
            return 0
        lax.fori_loop(ga, gb, fire, 0)

        def drain(g, _):
            pvec = plist[pl.ds(0, 16)]
            pltpu.make_async_copy(
                pf_hbm.at[pvec], rows.at[pl.ds(0, 16), :], sem_row).wait()
            return 0
        lax.fori_loop(ga, gb, drain, 0)

        def agrp(g, _):
            dvec = dlist[pl.ds(g * 16, 16)]

            def ldp(i):
                krow = (g - ga) * 16 + i
                return [rows[krow, pl.ds(cg * 16, 16)]
                        for cg in range(_C // 16)]
            vecs = ldp(0)
            for i in range(16):
                k = g * 16 + i
                m = jnp.broadcast_to((k >= seg0) & (k < seg1), (16,))
                dloc = dvec[i]
                nxt = ldp(i + 1) if i < 15 else None
                for cg in range(_C // 16):
                    plsc.store_scatter(
                        otile,
                        [jnp.full((16,), b, jnp.int32), cg * 16 + iota,
                         jnp.broadcast_to(dloc, (16,))],
                        vecs[cg], mask=m)
                vecs = nxt
            return 0
        lax.fori_loop(ga, gb, agrp, 0)

        pltpu.make_async_copy(
            otile.at[b, :, 0:_SUB],
            out_hbm.at[:, pl.ds(col0, _SUB)], sem_out).start()
        return (seg0, seg1)

    def pair(sp, carry):
        pa, pb = carry
        pa = half(sp * 2, 0, pa, sem_out0)
        pb = half(sp * 2 + 1, 1, pb, sem_out1)
        return (pa, pb)

    z = jnp.int32(0)
    lax.fori_loop(0, _NSUB // 2, pair, ((z, z), (z, z)))

    _scope_p3.__exit__(None, None, None)
    pltpu.make_async_copy(
        otile.at[0, :, 0:_SUB],
        out_hbm.at[:, pl.ds(base, _SUB)], sem_out0).wait()
    pltpu.make_async_copy(
        otile.at[1, :, 0:_SUB],
        out_hbm.at[:, pl.ds(base, _SUB)], sem_out1).wait()


_mesh = plsc.VectorSubcoreMesh(core_axis_name="c", subcore_axis_name="s")

_scatter = functools.partial(
    pl.kernel,
    out_type=jax.ShapeDtypeStruct((_C, _CELLS), jnp.float32),
    mesh=_mesh,
    compiler_params=pltpu.CompilerParams(use_tc_tiling_on_sc=False,
                                         needs_layout_passes=False),
    scratch_types=[
        pltpu.VMEM((_STRIPE,), jnp.int32),
        pltpu.VMEM((_CH1,), jnp.int32),
        pltpu.VMEM((_LISTCAP,), jnp.int32),
        pltpu.VMEM((_LISTCAP,), jnp.int32),
        pltpu.VMEM((_ROWCAP, _C), jnp.float32),
        pltpu.VMEM((2, _C, _OPAD), jnp.float32),
        pltpu.SMEM((_NSUB + 2,), jnp.int32),
        pltpu.SemaphoreType.DMA,
        pltpu.SemaphoreType.DMA,
        pltpu.SemaphoreType.DMA,
    ],
)(_body)


def kernel(pillar_features, coords):
    ci = coords.astype(jnp.int32)
    idx = ci[:, 1] * (_NY * _NX) + ci[:, 2] * _NX + ci[:, 3]
    pf = pillar_features.astype(jnp.float32)
    out = _scatter(idx, pf)
    return out.reshape(1, _C * _NZ, _NY, _NX)


# speedup vs baseline: 1.2480x; 1.0491x over previous
import functools

import jax
import jax.numpy as jnp
from jax import lax
from jax.experimental import pallas as pl
from jax.experimental.pallas import tpu as pltpu
from jax.experimental.pallas import tpu_sc as plsc

_NX, _NY, _NZ = 512, 512, 1
_C = 128
_P = 60000
_CELLS = _NZ * _NY * _NX
_NW = 32
_STRIPE = _CELLS // _NW
_SUB = 256
_NSUB = _STRIPE // _SUB
_CH1 = 2000
_NCH1 = _P // _CH1
_UNROLL1 = 5
_LISTCAP = _STRIPE + 16
_OPAD = 257
_ROWCAP = _SUB + 16


def _iota16():
    return lax.iota(jnp.int32, 16)


def _body(idx_hbm, pf_hbm, out_hbm,
          w_map, ibuf, plist, dlist, rows, otile, segs,
          sem_row, sem_out0, sem_out1):
    wid = lax.axis_index("s") * 2 + lax.axis_index("c")
    base = wid * _STRIPE
    iota = _iota16()
    zerosf = jnp.zeros((16,), jnp.float32)

    _scope_init = jax.named_scope("ph0_init"); _scope_init.__enter__()
    def initw(i, _):
        w_map[pl.ds(i * 16, 16)] = jnp.full((16,), -1, jnp.int32)
        return 0
    lax.fori_loop(0, _STRIPE // 16, initw, 0)

    for b in range(2):
        def inito(c, _):
            for cg in range(_SUB // 16):
                otile[b, c, pl.ds(cg * 16, 16)] = zerosf
            otile[b, c, pl.ds(_OPAD - 16, 16)] = zerosf
            return 0
        lax.fori_loop(0, _C, inito, 0)

    _scope_init.__exit__(None, None, None)
    _scope_p1 = jax.named_scope("ph1_winner"); _scope_p1.__enter__()
    def p1_chunk(ci, _):
        off = ci * _CH1
        pltpu.sync_copy(idx_hbm.at[pl.ds(off, _CH1)], ibuf)

        def grp(g, _):
            os_ = [g * (16 * _UNROLL1) + u * 16 for u in range(_UNROLL1)]
            idxs = [ibuf[pl.ds(o, 16)] for o in os_]
            scans = [plsc.scan_count(ix)[1] for ix in idxs]
            for o, ix, lastm in zip(os_, idxs, scans):
                lidx = ix - base
                inr = (lidx >= 0) & (lidx < _STRIPE)
                m = lastm & inr
                plsc.store_scatter(w_map, [lidx], off + o + iota, mask=m)
            return 0
        lax.fori_loop(0, _CH1 // (16 * _UNROLL1), grp, 0)
        return 0
    lax.fori_loop(0, _NCH1, p1_chunk, 0)

    _scope_p1.__exit__(None, None, None)
    _scope_p2 = jax.named_scope("ph2_compact"); _scope_p2.__enter__()
    def comp(s, cur):
        segs[s] = cur

        def cgrp(j, cur2):
            w = w_map[pl.ds(s * _SUB + j * 16, 16)]
            m = w >= 0
            plsc.store_compressed(plist.at[pl.ds(cur2, 16)], w, mask=m)
            plsc.store_compressed(dlist.at[pl.ds(cur2, 16)],
                                  j * 16 + iota, mask=m)
            return cur2 + plsc.all_reduce_population_count(m)[0]
        return lax.fori_loop(0, _SUB // 16, cgrp, cur)
    ktot = lax.fori_loop(0, _NSUB, comp, jnp.int32(0))
    segs[_NSUB] = ktot
    plsc.store_compressed(plist.at[pl.ds(ktot, 16)],
                          jnp.zeros((16,), jnp.int32),
                          mask=jnp.full((16,), True))
    plsc.store_compressed(dlist.at[pl.ds(ktot, 16)],
                          jnp.zeros((16,), jnp.int32),
                          mask=jnp.full((16,), True))

    _scope_p2.__exit__(None, None, None)
    _scope_p3 = jax.named_scope("ph3_assemble"); _scope_p3.__enter__()
    def half(s, b, pseg, sem_out):
        pseg0, pseg1 = pseg
        col0 = base + s * _SUB

        seg0 = segs[s]
        seg1 = segs[s + 1]
        ga = seg0 >> 4
        gb = (seg1 + 15) >> 4

        def fire(g, _):
            pvec = plist[pl.ds(g * 16, 16)]
            pltpu.make_async_copy(
                pf_hbm.at[pvec],
                rows.at[pl.ds((g - ga) * 16, 16), :], sem_row).start()
            return 0
        lax.fori_loop(ga, gb, fire, 0)

        @pl.when(s >= 2)
        def _wait_prev():
            pltpu.make_async_copy(
                otile.at[b, :, 0:_SUB],
                out_hbm.at[:, pl.ds(col0, _SUB)], sem_out).wait()

        def rgrp(g, _):
            dvec = dlist[pl.ds(g * 16, 16)]
            for i in range(16):
                dloc = dvec[i]
                for cg in range(_C // 16):
                    plsc.store_scatter(
                        otile,
                        [jnp.full((16,), b, jnp.int32), cg * 16 + iota,
                         jnp.broadcast_to(dloc, (16,))],
                        zerosf)
            return 0
        lax.fori_loop(pseg0 >> 4, (pseg1 + 15) >> 4, rgrp, 0)

        def drain(g, _):
            pvec = plist[pl.ds(0, 16)]
            pltpu.make_async_copy(
                pf_hbm.at[pvec], rows.at[pl.ds(0, 16), :], sem_row).wait()
            return 0
        lax.fori_loop(ga, gb, drain, 0)

        def agrp(g, _):
            dvec = dlist[pl.ds(g * 16, 16)]

            def ldp(i):
                krow = (g - ga) * 16 + i
                return [rows[krow, pl.ds(cg * 16, 16)]
                        for cg in range(_C // 16)]
            vecs = ldp(0)
            for i in range(16):
                k = g * 16 + i
                m = jnp.broadcast_to((k >= seg0) & (k < seg1), (16,))
                dloc = dvec[i]
                nxt = ldp(i + 1) if i < 15 else None
                for cg in range(_C // 16):
                    plsc.store_scatter(
                        otile,
                        [jnp.full((16,), b, jnp.int32), cg * 16 + iota,
                         jnp.broadcast_to(dloc, (16,))],
                        vecs[cg], mask=m)
                vecs = nxt
            return 0
        lax.fori_loop(ga, gb, agrp, 0)

        pltpu.make_async_copy(
            otile.at[b, :, 0:_SUB],
            out_hbm.at[:, pl.ds(col0, _SUB)], sem_out).start()
        return (seg0, seg1)

    def pair(sp, carry):
        pa, pb = carry
        pa = half(sp * 2, 0, pa, sem_out0)
        pb = half(sp * 2 + 1, 1, pb, sem_out1)
        return (pa, pb)

    z = jnp.int32(0)
    lax.fori_loop(0, _NSUB // 2, pair, ((z, z), (z, z)))

    _scope_p3.__exit__(None, None, None)
    pltpu.make_async_copy(
        otile.at[0, :, 0:_SUB],
        out_hbm.at[:, pl.ds(base, _SUB)], sem_out0).wait()
    pltpu.make_async_copy(
        otile.at[1, :, 0:_SUB],
        out_hbm.at[:, pl.ds(base, _SUB)], sem_out1).wait()


_mesh = plsc.VectorSubcoreMesh(core_axis_name="c", subcore_axis_name="s")

_scatter = functools.partial(
    pl.kernel,
    out_type=jax.ShapeDtypeStruct((_C, _CELLS), jnp.float32),
    mesh=_mesh,
    compiler_params=pltpu.CompilerParams(use_tc_tiling_on_sc=False,
                                         needs_layout_passes=False),
    scratch_types=[
        pltpu.VMEM((_STRIPE,), jnp.int32),
        pltpu.VMEM((_CH1,), jnp.int32),
        pltpu.VMEM((_LISTCAP,), jnp.int32),
        pltpu.VMEM((_LISTCAP,), jnp.int32),
        pltpu.VMEM((_ROWCAP, _C), jnp.float32),
        pltpu.VMEM((2, _C, _OPAD), jnp.float32),
        pltpu.SMEM((_NSUB + 2,), jnp.int32),
        pltpu.SemaphoreType.DMA,
        pltpu.SemaphoreType.DMA,
        pltpu.SemaphoreType.DMA,
    ],
)(_body)


def kernel(pillar_features, coords):
    ci = coords.astype(jnp.int32)
    idx = ci[:, 1] * (_NY * _NX) + ci[:, 2] * _NX + ci[:, 3]
    pf = pillar_features.astype(jnp.float32)
    out = _scatter(idx, pf)
    return out.reshape(1, _C * _NZ, _NY, _NX)
